# baseline (speedup 1.0000x reference)
import functools

import jax
import jax.numpy as jnp
from jax import lax
from jax.experimental import pallas as pl
from jax.experimental.pallas import tpu as pltpu
from jax.experimental.pallas import tpu_sc as plsc

EMBEDDING_DIM = 64
BATCH = 16384

_info = plsc.get_sparse_core_info()
_NC, _NS, _NL = _info.num_cores, _info.num_subcores, _info.num_lanes
_NW = _NC * _NS

_SC_BATCH = 8704
_TC_BATCH = BATCH - _SC_BATCH
_B_PER_W = _SC_BATCH // _NW
_N_CHUNKS = _B_PER_W // _NL

_mesh = plsc.VectorSubcoreMesh(core_axis_name="c", subcore_axis_name="s")


@functools.partial(
    pl.kernel,
    mesh=_mesh,
    out_type=jax.ShapeDtypeStruct((_SC_BATCH, EMBEDDING_DIM), jnp.float32),
    compiler_params=pltpu.CompilerParams(needs_layout_passes=False),
    cost_estimate=pl.CostEstimate(
        flops=0, transcendentals=0, bytes_accessed=600_000_000
    ),
    scratch_types=[
        pltpu.VMEM((_B_PER_W,), jnp.int32),
        pltpu.VMEM((_B_PER_W, EMBEDDING_DIM), jnp.float32),
        pltpu.SemaphoreType.DMA,
    ],
)
def _sc_gather(idx_hbm, table_hbm, out_hbm, idx_v, rows_v, sem):
    wid = lax.axis_index("s") * _NC + lax.axis_index("c")
    base = wid * _B_PER_W
    pltpu.sync_copy(idx_hbm.at[pl.ds(base, _B_PER_W)], idx_v)

    lane = lax.iota(jnp.int32, _NL)

    def fire(k, _):
        chunk = idx_v[pl.ds(k * _NL, _NL)]
        for j in range(_NL):
            r = jnp.sum(jnp.where(lane == j, chunk, 0))
            pltpu.async_copy(
                table_hbm.at[pl.ds(r, 1)],
                rows_v.at[pl.ds(k * _NL + j, 1)],
                sem,
            )
        return ()

    lax.fori_loop(0, _N_CHUNKS, fire, ())

    def drain(i, _):
        pltpu.make_async_copy(
            table_hbm.at[pl.ds(0, 1)], rows_v.at[pl.ds(0, 1)], sem
        ).wait()
        return ()

    lax.fori_loop(0, _B_PER_W, drain, ())
    pltpu.sync_copy(rows_v, out_hbm.at[pl.ds(base, _B_PER_W)])


def _tc_body(xs_smem, table_any, out_any, rows_v, sem, sem_out):
    def fire(i, _):
        r = xs_smem[i]
        pltpu.make_async_copy(
            table_any.at[pl.ds(r, 1)], rows_v.at[pl.ds(i, 1)], sem
        ).start()
        return ()

    lax.fori_loop(0, _TC_BATCH, fire, (), unroll=8)

    def drain(i, _):
        pltpu.make_async_copy(
            table_any.at[pl.ds(0, 1)], rows_v.at[pl.ds(0, 1)], sem
        ).wait()
        return ()

    lax.fori_loop(0, _TC_BATCH, drain, (), unroll=8)
    copy_out = pltpu.make_async_copy(rows_v, out_any, sem_out)
    copy_out.start()
    copy_out.wait()


_tc_gather = pl.pallas_call(
    _tc_body,
    grid=(),
    in_specs=[
        pl.BlockSpec(memory_space=pltpu.SMEM),
        pl.BlockSpec(memory_space=pl.ANY),
    ],
    out_specs=pl.BlockSpec(memory_space=pl.ANY),
    out_shape=jax.ShapeDtypeStruct((_TC_BATCH, EMBEDDING_DIM), jnp.float32),
    cost_estimate=pl.CostEstimate(
        flops=0, transcendentals=0, bytes_accessed=600_000_000
    ),
    scratch_shapes=[
        pltpu.VMEM((_TC_BATCH, EMBEDDING_DIM), jnp.float32),
        pltpu.SemaphoreType.DMA,
        pltpu.SemaphoreType.DMA,
    ],
)


def kernel(x, table):
    xi = x.astype(jnp.int32)
    out_sc = _sc_gather(xi[:_SC_BATCH], table)
    out_tc = _tc_gather(xi[_SC_BATCH:], table)
    return jnp.concatenate([out_sc, out_tc], axis=0)


# speedup vs baseline: 1.1087x; 1.1087x over previous
import functools

import jax
import jax.numpy as jnp
from jax import lax
from jax.experimental import pallas as pl
from jax.experimental.pallas import tpu as pltpu
from jax.experimental.pallas import tpu_sc as plsc

EMBEDDING_DIM = 64
BATCH = 16384
_NSEM = 4

_info = plsc.get_sparse_core_info()
_NC, _NS, _NL = _info.num_cores, _info.num_subcores, _info.num_lanes
_NW = _NC * _NS
_B_PER_W = BATCH // _NW
_N_CHUNKS = _B_PER_W // _NL

_mesh = plsc.VectorSubcoreMesh(core_axis_name="c", subcore_axis_name="s")


@functools.partial(
    pl.kernel,
    mesh=_mesh,
    out_type=jax.ShapeDtypeStruct((BATCH, EMBEDDING_DIM), jnp.float32),
    compiler_params=pltpu.CompilerParams(needs_layout_passes=False),
    scratch_types=[
        pltpu.VMEM((_B_PER_W,), jnp.int32),
        pltpu.VMEM((_B_PER_W, EMBEDDING_DIM), jnp.float32),
    ]
    + [pltpu.SemaphoreType.DMA] * _NSEM,
)
def _emb_lookup(idx_hbm, table_hbm, out_hbm, idx_v, rows_v, *sems):
    wid = lax.axis_index("s") * _NC + lax.axis_index("c")
    base = wid * _B_PER_W
    pltpu.sync_copy(idx_hbm.at[pl.ds(base, _B_PER_W)], idx_v)

    lane = lax.iota(jnp.int32, _NL)

    def fire(k, _):
        chunk = idx_v[pl.ds(k * _NL, _NL)]
        for j in range(_NL):
            r = jnp.sum(jnp.where(lane == j, chunk, 0))
            pltpu.async_copy(
                table_hbm.at[r], rows_v.at[k * _NL + j], sems[j % _NSEM]
            )
        return ()

    lax.fori_loop(0, _N_CHUNKS, fire, ())

    def drain(i, _):
        for s in range(_NSEM):
            pltpu.make_async_copy(
                table_hbm.at[0], rows_v.at[0], sems[s]
            ).wait()
        return ()

    lax.fori_loop(0, _B_PER_W // _NSEM, drain, ())
    pltpu.sync_copy(rows_v, out_hbm.at[pl.ds(base, _B_PER_W)])


def kernel(x, table):
    return _emb_lookup(x.astype(jnp.int32), table)
